# baseline (speedup 1.0000x reference)
import functools

import jax
import jax.numpy as jnp
from jax import lax
from jax.experimental import pallas as pl
from jax.experimental.pallas import tpu as pltpu
from jax.experimental.pallas import tpu_sc as plsc

N = 10000
E = 160000
NPAD = 10240
EPAD = 163840
NC, NS, L = 2, 16, 16
NW = NC * NS
EPW = EPAD // NW
CH = 128
NCH = EPW // CH
SEG = NPAD // NS
DH = 32
DL = 16
N_IN = 128

_MESH = plsc.VectorSubcoreMesh(
    core_axis_name="c", subcore_axis_name="s", num_cores=NC, num_subcores=NS)
_SC_PARAMS = pltpu.CompilerParams(
    needs_layout_passes=False, use_tc_tiling_on_sc=False)


@functools.partial(
    pl.kernel,
    out_type=jax.ShapeDtypeStruct((NC, NPAD), jnp.float32),
    mesh=_MESH,
    scratch_types=[
        pltpu.VMEM((NCH, CH), jnp.int32),
        pltpu.VMEM((NPAD,), jnp.float32),
        pltpu.VMEM((NS, SEG), jnp.float32),
        pltpu.VMEM((SEG,), jnp.float32),
        pltpu.VMEM_SHARED((NS, NPAD), jnp.float32),
        pltpu.SemaphoreType.DMA,
    ],
    compiler_params=_SC_PARAMS,
)
def _deg_kernel(dst_hbm, out_hbm, idx_v, hist_v, blk_v, red_v, shared, semi):
    cid = lax.axis_index("c")
    sid = lax.axis_index("s")
    wid = cid * NS + sid

    pltpu.async_copy(dst_hbm.at[wid], idx_v, semi)

    zeros = jnp.zeros((L,), jnp.float32)
    ones = jnp.ones((L,), jnp.float32)

    def _zero(i, _):
        hist_v[pl.ds(i * L, L)] = zeros
        return _
    lax.fori_loop(0, NPAD // L, _zero, None)

    pltpu.make_async_copy(dst_hbm.at[wid], idx_v, semi).wait()

    def _chunk(ci, _):
        def _vec(j, _):
            idx = idx_v[ci, pl.ds(j * L, L)]
            plsc.addupdate_scatter(hist_v, [idx], ones)
            return _
        lax.fori_loop(0, CH // L, _vec, None)
        return _
    lax.fori_loop(0, NCH, _chunk, None)

    pltpu.sync_copy(hist_v, shared.at[sid])
    plsc.subcore_barrier()

    seg = sid * SEG
    pltpu.sync_copy(shared.at[:, pl.ds(seg, SEG)], blk_v)

    def _reduce(i, _):
        acc = blk_v[0, pl.ds(i * L, L)]
        for r in range(1, NS):
            acc = acc + blk_v[r, pl.ds(i * L, L)]
        red_v[pl.ds(i * L, L)] = acc
        return _
    lax.fori_loop(0, SEG // L, _reduce, None)

    pltpu.sync_copy(red_v, out_hbm.at[cid, pl.ds(seg, SEG)])


@functools.partial(
    pl.kernel,
    out_type=jax.ShapeDtypeStruct((NC, NPAD, DH), jnp.float32),
    mesh=_MESH,
    scratch_types=[
        pltpu.VMEM((NCH, CH), jnp.int32),
        pltpu.VMEM((NCH, CH), jnp.int32),
        pltpu.VMEM((CH, DH), jnp.float32),
        pltpu.VMEM((CH, DH), jnp.float32),
        pltpu.VMEM((SEG, DH), jnp.float32),
        pltpu.VMEM_SHARED((NPAD, DH), jnp.float32),
        pltpu.SemaphoreType.DMA,
        pltpu.SemaphoreType.DMA,
        pltpu.SemaphoreType.DMA,
    ],
    compiler_params=_SC_PARAMS,
)
def _spmm_kernel(u_hbm, src_hbm, dst_hbm, out_hbm, src_all, dst_all,
                 rows0, rows1, wb, acc, sem0, sem1, semi):
    cid = lax.axis_index("c")
    sid = lax.axis_index("s")
    wid = cid * NS + sid

    pltpu.async_copy(src_hbm.at[wid], src_all, semi)
    pltpu.async_copy(dst_hbm.at[wid], dst_all, semi)

    zeros = jnp.zeros((L,), jnp.float32)

    def _zrow(i, _):
        wb[i, pl.ds(0, L)] = zeros
        wb[i, pl.ds(L, L)] = zeros
        return _
    lax.fori_loop(0, SEG, _zrow, None)
    pltpu.sync_copy(wb, acc.at[pl.ds(sid * SEG, SEG), :])

    pltpu.make_async_copy(src_hbm.at[wid], src_all, semi).wait()
    pltpu.make_async_copy(dst_hbm.at[wid], dst_all, semi).wait()
    pltpu.async_copy(u_hbm.at[src_all.at[0]], rows0, sem0)
    plsc.subcore_barrier()

    def _pipe(j, _):
        c0 = 2 * j
        c1 = c0 + 1
        c2 = jnp.minimum(c0 + 2, NCH - 2)
        pltpu.make_async_copy(u_hbm.at[src_all.at[c0]], rows0, sem0).wait()
        pltpu.async_copy(u_hbm.at[src_all.at[c1]], rows1, sem1)
        pltpu.sync_copy(rows0, acc.at[dst_all.at[c0]], add=True)
        pltpu.make_async_copy(u_hbm.at[src_all.at[c1]], rows1, sem1).wait()
        pltpu.async_copy(u_hbm.at[src_all.at[c2]], rows0, sem0)
        pltpu.sync_copy(rows1, acc.at[dst_all.at[c1]], add=True)
        return _
    lax.fori_loop(0, NCH // 2, _pipe, None)
    pltpu.make_async_copy(u_hbm.at[src_all.at[0]], rows0, sem0).wait()

    plsc.subcore_barrier()
    pltpu.sync_copy(acc.at[pl.ds(sid * SEG, SEG), :], wb)
    pltpu.sync_copy(wb, out_hbm.at[cid, pl.ds(sid * SEG, SEG), :])


_RB = 2000


def _xw_body(x_ref, w_ref, xw_ref):
    xw_ref[...] = jnp.dot(
        x_ref[...], w_ref[...], preferred_element_type=jnp.float32)


_xw = pl.pallas_call(
    _xw_body,
    out_shape=jax.ShapeDtypeStruct((N, DH), jnp.float32),
)


def _proj_body(xw_ref, d_ref, u_ref, dinv_ref):
    dinv = lax.rsqrt(d_ref[0] + d_ref[1] + 1.0)
    u_ref[...] = xw_ref[...] * dinv
    dinv_ref[...] = dinv


_proj1 = pl.pallas_call(
    _proj_body,
    grid=(1,),
    in_specs=[
        pl.BlockSpec((N, DH), lambda i: (0, 0)),
        pl.BlockSpec((NC, N, 1), lambda i: (0, 0, 0)),
    ],
    out_specs=(pl.BlockSpec((N, DH), lambda i: (0, 0)),
               pl.BlockSpec((N, 1), lambda i: (0, 0))),
    out_shape=(jax.ShapeDtypeStruct((N, DH), jnp.float32),
               jax.ShapeDtypeStruct((N, 1), jnp.float32)),
)


def _enc_body(s_ref, u1_ref, dinv_ref, b1_ref, wcat_ref, u2_ref):
    dinv = dinv_ref[...]
    h = s_ref[0] + s_ref[1] + u1_ref[...]
    h = jnp.maximum(h * dinv + b1_ref[...], 0.0)
    hm = jnp.dot(h, wcat_ref[...], preferred_element_type=jnp.float32)
    u2_ref[...] = hm * dinv


_enc = pl.pallas_call(
    _enc_body,
    grid=(1,),
    in_specs=[
        pl.BlockSpec((NC, N, DH), lambda i: (0, 0, 0)),
        pl.BlockSpec((N, DH), lambda i: (0, 0)),
        pl.BlockSpec((N, 1), lambda i: (0, 0)),
        pl.BlockSpec((1, DH), lambda i: (0, 0)),
        pl.BlockSpec((DH, DH), lambda i: (0, 0)),
    ],
    out_specs=pl.BlockSpec((N, DH), lambda i: (0, 0)),
    out_shape=jax.ShapeDtypeStruct((N, DH), jnp.float32),
)


def _head_body(s_ref, u2_ref, dinv_ref, bcat_ref, eps_ref,
               mu_ref, ls_ref, z_ref):
    g = (s_ref[0] + s_ref[1] + u2_ref[...]) * dinv_ref[...]
    g = g + bcat_ref[...]
    mu = g[:, :DL]
    ls = g[:, DL:]
    mu_ref[...] = mu
    ls_ref[...] = ls
    z_ref[...] = mu + eps_ref[...] * jnp.exp(ls)


_head = pl.pallas_call(
    _head_body,
    grid=(1,),
    in_specs=[
        pl.BlockSpec((NC, N, DH), lambda i: (0, 0, 0)),
        pl.BlockSpec((N, DH), lambda i: (0, 0)),
        pl.BlockSpec((N, 1), lambda i: (0, 0)),
        pl.BlockSpec((1, DH), lambda i: (0, 0)),
        pl.BlockSpec((N, DL), lambda i: (0, 0)),
    ],
    out_specs=(pl.BlockSpec((N, DL), lambda i: (0, 0)),
               pl.BlockSpec((N, DL), lambda i: (0, 0)),
               pl.BlockSpec((N, DL), lambda i: (0, 0))),
    out_shape=(jax.ShapeDtypeStruct((N, DL), jnp.float32),
               jax.ShapeDtypeStruct((N, DL), jnp.float32),
               jax.ShapeDtypeStruct((N, DL), jnp.float32)),
)


_BR, _BC = 400, 10000


def _dec_body(zr_ref, zc_ref, o_ref):
    o_ref[...] = lax.dot_general(
        zr_ref[...], zc_ref[...], (((1,), (1,)), ((), ())),
        preferred_element_type=jnp.float32)


_decoder = pl.pallas_call(
    _dec_body,
    grid=(N // _BR,),
    in_specs=[
        pl.BlockSpec((_BR, DL), lambda i: (i, 0)),
        pl.BlockSpec((_BC, DL), lambda i: (0, 0)),
    ],
    out_specs=pl.BlockSpec((_BR, _BC), lambda i: (i, 0)),
    out_shape=jax.ShapeDtypeStruct((N, N), jnp.float32),
)


def kernel(x, W1, b1, Wmu, bmu, Wls, bls, edge_index):
    src = edge_index[0]
    dst = edge_index[1]
    npad = EPAD - E
    src_p = jnp.concatenate(
        [src, jnp.zeros((npad,), jnp.int32)]).reshape(NW, NCH, CH)
    dst_p = jnp.concatenate(
        [dst, jnp.full((npad,), N, jnp.int32)]).reshape(NW, NCH, CH)

    xw = _xw(x, W1)
    degs = _deg_kernel(dst_p).reshape(NC, NPAD, 1)

    u1, dinv = _proj1(xw, degs)

    s1 = _spmm_kernel(u1, src_p, dst_p)
    wcat = jnp.concatenate([Wmu, Wls], axis=1)
    u2 = _enc(s1, u1, dinv, b1.reshape(1, DH), wcat)

    s2 = _spmm_kernel(u2, src_p, dst_p)
    bcat = jnp.concatenate([bmu, bls]).reshape(1, DH)
    eps = jax.random.normal(jax.random.key(42), (N, DL), dtype=jnp.float32)
    mu, logstd, z = _head(s2, u2, dinv, bcat, eps)
    adj = _decoder(z, z)
    return adj, mu, logstd


# speedup vs baseline: 1.0010x; 1.0010x over previous
import functools

import jax
import jax.numpy as jnp
from jax import lax
from jax.experimental import pallas as pl
from jax.experimental.pallas import tpu as pltpu
from jax.experimental.pallas import tpu_sc as plsc

N = 10000
E = 160000
NPAD = 10240
EPAD = 163840
NC, NS, L = 2, 16, 16
NW = NC * NS
EPW = EPAD // NW
CH = 128
NCH = EPW // CH
SEG = NPAD // NS
DH = 32
DL = 16
N_IN = 128

_MESH = plsc.VectorSubcoreMesh(
    core_axis_name="c", subcore_axis_name="s", num_cores=NC, num_subcores=NS)
_SC_PARAMS = pltpu.CompilerParams(
    needs_layout_passes=False, use_tc_tiling_on_sc=False)


@functools.partial(
    pl.kernel,
    out_type=jax.ShapeDtypeStruct((NC, NPAD), jnp.float32),
    mesh=_MESH,
    scratch_types=[
        pltpu.VMEM((NCH, CH), jnp.int32),
        pltpu.VMEM((NPAD,), jnp.float32),
        pltpu.VMEM((NS, SEG), jnp.float32),
        pltpu.VMEM((SEG,), jnp.float32),
        pltpu.VMEM_SHARED((NS, NPAD), jnp.float32),
        pltpu.SemaphoreType.DMA,
    ],
    compiler_params=_SC_PARAMS,
)
def _deg_kernel(dst_hbm, out_hbm, idx_v, hist_v, blk_v, red_v, shared, semi):
    cid = lax.axis_index("c")
    sid = lax.axis_index("s")
    wid = cid * NS + sid

    pltpu.async_copy(dst_hbm.at[wid], idx_v, semi)

    zeros = jnp.zeros((L,), jnp.float32)
    ones = jnp.ones((L,), jnp.float32)

    def _zero(i, _):
        hist_v[pl.ds(i * L, L)] = zeros
        return _
    lax.fori_loop(0, NPAD // L, _zero, None)

    pltpu.make_async_copy(dst_hbm.at[wid], idx_v, semi).wait()

    def _chunk(ci, _):
        def _vec(j, _):
            idx = idx_v[ci, pl.ds(j * L, L)]
            plsc.addupdate_scatter(hist_v, [idx], ones)
            return _
        lax.fori_loop(0, CH // L, _vec, None)
        return _
    lax.fori_loop(0, NCH, _chunk, None)

    pltpu.sync_copy(hist_v, shared.at[sid])
    plsc.subcore_barrier()

    seg = sid * SEG
    pltpu.sync_copy(shared.at[:, pl.ds(seg, SEG)], blk_v)

    def _reduce(i, _):
        acc = blk_v[0, pl.ds(i * L, L)]
        for r in range(1, NS):
            acc = acc + blk_v[r, pl.ds(i * L, L)]
        red_v[pl.ds(i * L, L)] = acc
        return _
    lax.fori_loop(0, SEG // L, _reduce, None)

    pltpu.sync_copy(red_v, out_hbm.at[cid, pl.ds(seg, SEG)])


@functools.partial(
    pl.kernel,
    out_type=jax.ShapeDtypeStruct((NC, NPAD, DH), jnp.float32),
    mesh=_MESH,
    scratch_types=[
        pltpu.VMEM((NCH, CH), jnp.int32),
        pltpu.VMEM((NCH, CH), jnp.int32),
        pltpu.VMEM((CH, DH), jnp.float32),
        pltpu.VMEM((CH, DH), jnp.float32),
        pltpu.VMEM((SEG, DH), jnp.float32),
        pltpu.VMEM_SHARED((NPAD, DH), jnp.float32),
        pltpu.SemaphoreType.DMA,
        pltpu.SemaphoreType.DMA,
        pltpu.SemaphoreType.DMA,
    ],
    compiler_params=_SC_PARAMS,
)
def _spmm_kernel(u_hbm, src_hbm, dst_hbm, out_hbm, src_all, dst_all,
                 rows0, rows1, wb, acc, sem0, sem1, semi):
    cid = lax.axis_index("c")
    sid = lax.axis_index("s")
    wid = cid * NS + sid

    pltpu.async_copy(src_hbm.at[wid], src_all, semi)
    pltpu.async_copy(dst_hbm.at[wid], dst_all, semi)

    zeros = jnp.zeros((L,), jnp.float32)

    def _zrow(i, _):
        wb[i, pl.ds(0, L)] = zeros
        wb[i, pl.ds(L, L)] = zeros
        return _
    lax.fori_loop(0, SEG, _zrow, None)
    pltpu.sync_copy(wb, acc.at[pl.ds(sid * SEG, SEG), :])

    pltpu.make_async_copy(src_hbm.at[wid], src_all, semi).wait()
    pltpu.make_async_copy(dst_hbm.at[wid], dst_all, semi).wait()
    pltpu.async_copy(u_hbm.at[src_all.at[0]], rows0, sem0)
    plsc.subcore_barrier()

    def _pipe(j, _):
        c0 = 2 * j
        c1 = c0 + 1
        c2 = jnp.minimum(c0 + 2, NCH - 2)
        pltpu.make_async_copy(u_hbm.at[src_all.at[c0]], rows0, sem0).wait()
        pltpu.async_copy(u_hbm.at[src_all.at[c1]], rows1, sem1)
        pltpu.sync_copy(rows0, acc.at[dst_all.at[c0]], add=True)
        pltpu.make_async_copy(u_hbm.at[src_all.at[c1]], rows1, sem1).wait()
        pltpu.async_copy(u_hbm.at[src_all.at[c2]], rows0, sem0)
        pltpu.sync_copy(rows1, acc.at[dst_all.at[c1]], add=True)
        return _
    lax.fori_loop(0, NCH // 2, _pipe, None)
    pltpu.make_async_copy(u_hbm.at[src_all.at[0]], rows0, sem0).wait()

    plsc.subcore_barrier()
    pltpu.sync_copy(acc.at[pl.ds(sid * SEG, SEG), :], wb)
    pltpu.sync_copy(wb, out_hbm.at[cid, pl.ds(sid * SEG, SEG), :])


_RB = 2000


def _proj_body(x_ref, w_ref, d_ref, u_ref, dinv_ref):
    dinv = lax.rsqrt(d_ref[0] + d_ref[1] + 1.0)
    xw = jnp.dot(x_ref[...], w_ref[...], preferred_element_type=jnp.float32)
    u_ref[...] = xw * dinv
    dinv_ref[...] = dinv


_proj1 = pl.pallas_call(
    _proj_body,
    grid=(1,),
    in_specs=[
        pl.BlockSpec((N, N_IN), lambda i: (0, 0)),
        pl.BlockSpec((N_IN, DH), lambda i: (0, 0)),
        pl.BlockSpec((NC, N, 1), lambda i: (0, 0, 0)),
    ],
    out_specs=(pl.BlockSpec((N, DH), lambda i: (0, 0)),
               pl.BlockSpec((N, 1), lambda i: (0, 0))),
    out_shape=(jax.ShapeDtypeStruct((N, DH), jnp.float32),
               jax.ShapeDtypeStruct((N, 1), jnp.float32)),
)


def _enc_body(s_ref, u1_ref, dinv_ref, b1_ref, wcat_ref, u2_ref):
    dinv = dinv_ref[...]
    h = s_ref[0] + s_ref[1] + u1_ref[...]
    h = jnp.maximum(h * dinv + b1_ref[...], 0.0)
    hm = jnp.dot(h, wcat_ref[...], preferred_element_type=jnp.float32)
    u2_ref[...] = hm * dinv


_enc = pl.pallas_call(
    _enc_body,
    grid=(1,),
    in_specs=[
        pl.BlockSpec((NC, N, DH), lambda i: (0, 0, 0)),
        pl.BlockSpec((N, DH), lambda i: (0, 0)),
        pl.BlockSpec((N, 1), lambda i: (0, 0)),
        pl.BlockSpec((1, DH), lambda i: (0, 0)),
        pl.BlockSpec((DH, DH), lambda i: (0, 0)),
    ],
    out_specs=pl.BlockSpec((N, DH), lambda i: (0, 0)),
    out_shape=jax.ShapeDtypeStruct((N, DH), jnp.float32),
)


def _head_body(s_ref, u2_ref, dinv_ref, bcat_ref, eps_ref,
               mu_ref, ls_ref, z_ref):
    g = (s_ref[0] + s_ref[1] + u2_ref[...]) * dinv_ref[...]
    g = g + bcat_ref[...]
    mu = g[:, :DL]
    ls = g[:, DL:]
    mu_ref[...] = mu
    ls_ref[...] = ls
    z_ref[...] = mu + eps_ref[...] * jnp.exp(ls)


_head = pl.pallas_call(
    _head_body,
    grid=(1,),
    in_specs=[
        pl.BlockSpec((NC, N, DH), lambda i: (0, 0, 0)),
        pl.BlockSpec((N, DH), lambda i: (0, 0)),
        pl.BlockSpec((N, 1), lambda i: (0, 0)),
        pl.BlockSpec((1, DH), lambda i: (0, 0)),
        pl.BlockSpec((N, DL), lambda i: (0, 0)),
    ],
    out_specs=(pl.BlockSpec((N, DL), lambda i: (0, 0)),
               pl.BlockSpec((N, DL), lambda i: (0, 0)),
               pl.BlockSpec((N, DL), lambda i: (0, 0))),
    out_shape=(jax.ShapeDtypeStruct((N, DL), jnp.float32),
               jax.ShapeDtypeStruct((N, DL), jnp.float32),
               jax.ShapeDtypeStruct((N, DL), jnp.float32)),
)


_BR, _BC = 400, 10000


def _dec_body(zr_ref, zc_ref, o_ref):
    o_ref[...] = lax.dot_general(
        zr_ref[...], zc_ref[...], (((1,), (1,)), ((), ())),
        preferred_element_type=jnp.float32)


_decoder = pl.pallas_call(
    _dec_body,
    grid=(N // _BR,),
    in_specs=[
        pl.BlockSpec((_BR, DL), lambda i: (i, 0)),
        pl.BlockSpec((_BC, DL), lambda i: (0, 0)),
    ],
    out_specs=pl.BlockSpec((_BR, _BC), lambda i: (i, 0)),
    out_shape=jax.ShapeDtypeStruct((N, N), jnp.float32),
)


def kernel(x, W1, b1, Wmu, bmu, Wls, bls, edge_index):
    src = edge_index[0]
    dst = edge_index[1]
    npad = EPAD - E
    src_p = jnp.concatenate(
        [src, jnp.zeros((npad,), jnp.int32)]).reshape(NW, NCH, CH)
    dst_p = jnp.concatenate(
        [dst, jnp.full((npad,), N, jnp.int32)]).reshape(NW, NCH, CH)

    degs = _deg_kernel(dst_p).reshape(NC, NPAD, 1)

    u1, dinv = _proj1(x, W1, degs)

    s1 = _spmm_kernel(u1, src_p, dst_p)
    wcat = jnp.concatenate([Wmu, Wls], axis=1)
    u2 = _enc(s1, u1, dinv, b1.reshape(1, DH), wcat)

    s2 = _spmm_kernel(u2, src_p, dst_p)
    bcat = jnp.concatenate([bmu, bls]).reshape(1, DH)
    eps = jax.random.normal(jax.random.key(42), (N, DL), dtype=jnp.float32)
    mu, logstd, z = _head(s2, u2, dinv, bcat, eps)
    adj = _decoder(z, z)
    return adj, mu, logstd


# speedup vs baseline: 1.0206x; 1.0195x over previous
import functools

import jax
import jax.numpy as jnp
from jax import lax
from jax.experimental import pallas as pl
from jax.experimental.pallas import tpu as pltpu
from jax.experimental.pallas import tpu_sc as plsc

N = 10000
E = 160000
NPAD = 10240
EPAD = 163840
NC, NS, L = 2, 16, 16
NW = NC * NS
EPW = EPAD // NW
CH = 128
NCH = EPW // CH
SEG = NPAD // NS
DH = 32
DL = 16
N_IN = 128

_MESH = plsc.VectorSubcoreMesh(
    core_axis_name="c", subcore_axis_name="s", num_cores=NC, num_subcores=NS)
_SC_PARAMS = pltpu.CompilerParams(
    needs_layout_passes=False, use_tc_tiling_on_sc=False)


_EW = E // NW


@functools.partial(
    pl.kernel,
    out_type=jax.ShapeDtypeStruct((NC, NPAD), jnp.float32),
    mesh=_MESH,
    scratch_types=[
        pltpu.VMEM((_EW + L, ), jnp.int32),
        pltpu.VMEM((NPAD,), jnp.float32),
        pltpu.VMEM((NS, SEG), jnp.float32),
        pltpu.VMEM((SEG,), jnp.float32),
        pltpu.VMEM_SHARED((NS, NPAD), jnp.float32),
        pltpu.SemaphoreType.DMA,
    ],
    compiler_params=_SC_PARAMS,
)
def _deg_kernel(dst_hbm, out_hbm, idx_v, hist_v, blk_v, red_v, shared, semi):
    cid = lax.axis_index("c")
    sid = lax.axis_index("s")
    wid = cid * NS + sid

    pltpu.async_copy(dst_hbm.at[pl.ds(wid * _EW, _EW)],
                     idx_v.at[pl.ds(0, _EW)], semi)

    zeros = jnp.zeros((L,), jnp.float32)
    ones = jnp.ones((L,), jnp.float32)

    def _zero(i, _):
        hist_v[pl.ds(i * L, L)] = zeros
        return _
    lax.fori_loop(0, NPAD // L, _zero, None)

    pltpu.make_async_copy(dst_hbm.at[pl.ds(wid * _EW, _EW)],
                          idx_v.at[pl.ds(0, _EW)], semi).wait()

    def _vec(j, _):
        idx = idx_v[pl.ds(j * L, L)]
        plsc.addupdate_scatter(hist_v, [idx], ones)
        return _
    lax.fori_loop(0, _EW // L, _vec, None)
    tail = idx_v[pl.ds((_EW // L) * L, L)]
    lane = lax.iota(jnp.int32, L)
    tail = jnp.where(lane < (_EW % L), tail, N)
    plsc.addupdate_scatter(hist_v, [tail], ones)

    pltpu.sync_copy(hist_v, shared.at[sid])
    plsc.subcore_barrier()

    seg = sid * SEG
    pltpu.sync_copy(shared.at[:, pl.ds(seg, SEG)], blk_v)

    def _reduce(i, _):
        acc = blk_v[0, pl.ds(i * L, L)]
        for r in range(1, NS):
            acc = acc + blk_v[r, pl.ds(i * L, L)]
        red_v[pl.ds(i * L, L)] = acc
        return _
    lax.fori_loop(0, SEG // L, _reduce, None)

    pltpu.sync_copy(red_v, out_hbm.at[cid, pl.ds(seg, SEG)])


@functools.partial(
    pl.kernel,
    out_type=jax.ShapeDtypeStruct((NC, NPAD, DH), jnp.float32),
    mesh=_MESH,
    scratch_types=[
        pltpu.VMEM((NCH, CH), jnp.int32),
        pltpu.VMEM((NCH, CH), jnp.int32),
        pltpu.VMEM((CH, DH), jnp.float32),
        pltpu.VMEM((CH, DH), jnp.float32),
        pltpu.VMEM((SEG, DH), jnp.float32),
        pltpu.VMEM_SHARED((NPAD, DH), jnp.float32),
        pltpu.SemaphoreType.DMA,
        pltpu.SemaphoreType.DMA,
        pltpu.SemaphoreType.DMA,
    ],
    compiler_params=_SC_PARAMS,
)
def _spmm_kernel(u_hbm, src_hbm, dst_hbm, out_hbm, src_all, dst_all,
                 rows0, rows1, wb, acc, sem0, sem1, semi):
    cid = lax.axis_index("c")
    sid = lax.axis_index("s")
    wid = cid * NS + sid

    pltpu.async_copy(src_hbm.at[wid], src_all, semi)
    pltpu.async_copy(dst_hbm.at[wid], dst_all, semi)

    zeros = jnp.zeros((L,), jnp.float32)

    def _zrow(i, _):
        wb[i, pl.ds(0, L)] = zeros
        wb[i, pl.ds(L, L)] = zeros
        return _
    lax.fori_loop(0, SEG, _zrow, None)
    pltpu.sync_copy(wb, acc.at[pl.ds(sid * SEG, SEG), :])

    pltpu.make_async_copy(src_hbm.at[wid], src_all, semi).wait()
    pltpu.make_async_copy(dst_hbm.at[wid], dst_all, semi).wait()
    pltpu.async_copy(u_hbm.at[src_all.at[0]], rows0, sem0)
    plsc.subcore_barrier()

    def _pipe(j, _):
        c0 = 2 * j
        c1 = c0 + 1
        c2 = jnp.minimum(c0 + 2, NCH - 2)
        pltpu.make_async_copy(u_hbm.at[src_all.at[c0]], rows0, sem0).wait()
        pltpu.async_copy(u_hbm.at[src_all.at[c1]], rows1, sem1)
        pltpu.sync_copy(rows0, acc.at[dst_all.at[c0]], add=True)
        pltpu.make_async_copy(u_hbm.at[src_all.at[c1]], rows1, sem1).wait()
        pltpu.async_copy(u_hbm.at[src_all.at[c2]], rows0, sem0)
        pltpu.sync_copy(rows1, acc.at[dst_all.at[c1]], add=True)
        return _
    lax.fori_loop(0, NCH // 2, _pipe, None)
    pltpu.make_async_copy(u_hbm.at[src_all.at[0]], rows0, sem0).wait()

    plsc.subcore_barrier()
    pltpu.sync_copy(acc.at[pl.ds(sid * SEG, SEG), :], wb)
    pltpu.sync_copy(wb, out_hbm.at[cid, pl.ds(sid * SEG, SEG), :])


_RB = 2000


def _proj_body(x_ref, w_ref, d_ref, u_ref, dinv_ref):
    dinv = lax.rsqrt(d_ref[0] + d_ref[1] + 1.0)
    xw = jnp.dot(x_ref[...], w_ref[...], preferred_element_type=jnp.float32)
    u_ref[...] = xw * dinv
    dinv_ref[...] = dinv


_proj1 = pl.pallas_call(
    _proj_body,
    grid=(1,),
    in_specs=[
        pl.BlockSpec((N, N_IN), lambda i: (0, 0)),
        pl.BlockSpec((N_IN, DH), lambda i: (0, 0)),
        pl.BlockSpec((NC, N, 1), lambda i: (0, 0, 0)),
    ],
    out_specs=(pl.BlockSpec((N, DH), lambda i: (0, 0)),
               pl.BlockSpec((N, 1), lambda i: (0, 0))),
    out_shape=(jax.ShapeDtypeStruct((N, DH), jnp.float32),
               jax.ShapeDtypeStruct((N, 1), jnp.float32)),
)


def _enc_body(s_ref, u1_ref, dinv_ref, b1_ref, wcat_ref, u2_ref):
    dinv = dinv_ref[...]
    h = s_ref[0] + s_ref[1] + u1_ref[...]
    h = jnp.maximum(h * dinv + b1_ref[...], 0.0)
    hm = jnp.dot(h, wcat_ref[...], preferred_element_type=jnp.float32)
    u2_ref[...] = hm * dinv


_enc = pl.pallas_call(
    _enc_body,
    grid=(1,),
    in_specs=[
        pl.BlockSpec((NC, N, DH), lambda i: (0, 0, 0)),
        pl.BlockSpec((N, DH), lambda i: (0, 0)),
        pl.BlockSpec((N, 1), lambda i: (0, 0)),
        pl.BlockSpec((1, DH), lambda i: (0, 0)),
        pl.BlockSpec((DH, DH), lambda i: (0, 0)),
    ],
    out_specs=pl.BlockSpec((N, DH), lambda i: (0, 0)),
    out_shape=jax.ShapeDtypeStruct((N, DH), jnp.float32),
)


def _head_body(s_ref, u2_ref, dinv_ref, bcat_ref, eps_ref,
               mu_ref, ls_ref, z_ref):
    g = (s_ref[0] + s_ref[1] + u2_ref[...]) * dinv_ref[...]
    g = g + bcat_ref[...]
    mu = g[:, :DL]
    ls = g[:, DL:]
    mu_ref[...] = mu
    ls_ref[...] = ls
    z_ref[...] = mu + eps_ref[...] * jnp.exp(ls)


_head = pl.pallas_call(
    _head_body,
    grid=(1,),
    in_specs=[
        pl.BlockSpec((NC, N, DH), lambda i: (0, 0, 0)),
        pl.BlockSpec((N, DH), lambda i: (0, 0)),
        pl.BlockSpec((N, 1), lambda i: (0, 0)),
        pl.BlockSpec((1, DH), lambda i: (0, 0)),
        pl.BlockSpec((N, DL), lambda i: (0, 0)),
    ],
    out_specs=(pl.BlockSpec((N, DL), lambda i: (0, 0)),
               pl.BlockSpec((N, DL), lambda i: (0, 0)),
               pl.BlockSpec((N, DL), lambda i: (0, 0))),
    out_shape=(jax.ShapeDtypeStruct((N, DL), jnp.float32),
               jax.ShapeDtypeStruct((N, DL), jnp.float32),
               jax.ShapeDtypeStruct((N, DL), jnp.float32)),
)


_BR, _BC = 400, 10000


def _dec_body(zr_ref, zc_ref, o_ref):
    o_ref[...] = lax.dot_general(
        zr_ref[...], zc_ref[...], (((1,), (1,)), ((), ())),
        preferred_element_type=jnp.float32)


_decoder = pl.pallas_call(
    _dec_body,
    grid=(N // _BR,),
    in_specs=[
        pl.BlockSpec((_BR, DL), lambda i: (i, 0)),
        pl.BlockSpec((_BC, DL), lambda i: (0, 0)),
    ],
    out_specs=pl.BlockSpec((_BR, _BC), lambda i: (i, 0)),
    out_shape=jax.ShapeDtypeStruct((N, N), jnp.float32),
)


def kernel(x, W1, b1, Wmu, bmu, Wls, bls, edge_index):
    src = edge_index[0]
    dst = edge_index[1]
    npad = EPAD - E
    src_p = jnp.concatenate(
        [src, jnp.zeros((npad,), jnp.int32)]).reshape(NW, NCH, CH)
    dst_p = jnp.concatenate(
        [dst, jnp.full((npad,), N, jnp.int32)]).reshape(NW, NCH, CH)

    degs = _deg_kernel(dst).reshape(NC, NPAD, 1)

    u1, dinv = _proj1(x, W1, degs)

    s1 = _spmm_kernel(u1, src_p, dst_p)
    wcat = jnp.concatenate([Wmu, Wls], axis=1)
    u2 = _enc(s1, u1, dinv, b1.reshape(1, DH), wcat)

    s2 = _spmm_kernel(u2, src_p, dst_p)
    bcat = jnp.concatenate([bmu, bls]).reshape(1, DH)
    eps = jax.random.normal(jax.random.key(42), (N, DL), dtype=jnp.float32)
    mu, logstd, z = _head(s2, u2, dinv, bcat, eps)
    adj = _decoder(z, z)
    return adj, mu, logstd


# speedup vs baseline: 1.0206x; 1.0000x over previous
import functools

import jax
import jax.numpy as jnp
from jax import lax
from jax.experimental import pallas as pl
from jax.experimental.pallas import tpu as pltpu
from jax.experimental.pallas import tpu_sc as plsc

N = 10000
E = 160000
NPAD = 10240
EPAD = 163840
NC, NS, L = 2, 16, 16
NW = NC * NS
EPW = EPAD // NW
CH = 128
NCH = EPW // CH
SEG = NPAD // NS
DH = 32
DL = 16
N_IN = 128

_MESH = plsc.VectorSubcoreMesh(
    core_axis_name="c", subcore_axis_name="s", num_cores=NC, num_subcores=NS)
_SC_PARAMS = pltpu.CompilerParams(
    needs_layout_passes=False, use_tc_tiling_on_sc=False)


_EW = E // NW


@functools.partial(
    pl.kernel,
    out_type=jax.ShapeDtypeStruct((NC, NPAD), jnp.float32),
    mesh=_MESH,
    scratch_types=[
        pltpu.VMEM((_EW + L, ), jnp.int32),
        pltpu.VMEM((NPAD,), jnp.float32),
        pltpu.VMEM((NS, SEG), jnp.float32),
        pltpu.VMEM((SEG,), jnp.float32),
        pltpu.VMEM_SHARED((NS, NPAD), jnp.float32),
        pltpu.SemaphoreType.DMA,
    ],
    compiler_params=_SC_PARAMS,
)
def _deg_kernel(dst_hbm, out_hbm, idx_v, hist_v, blk_v, red_v, shared, semi):
    cid = lax.axis_index("c")
    sid = lax.axis_index("s")
    wid = cid * NS + sid

    pltpu.async_copy(dst_hbm.at[pl.ds(wid * _EW, _EW)],
                     idx_v.at[pl.ds(0, _EW)], semi)

    zeros = jnp.zeros((L,), jnp.float32)
    ones = jnp.ones((L,), jnp.float32)

    def _zero(i, _):
        hist_v[pl.ds(i * L, L)] = zeros
        return _
    lax.fori_loop(0, NPAD // L, _zero, None)

    pltpu.make_async_copy(dst_hbm.at[pl.ds(wid * _EW, _EW)],
                          idx_v.at[pl.ds(0, _EW)], semi).wait()

    def _vec(j, _):
        idx = idx_v[pl.ds(j * L, L)]
        plsc.addupdate_scatter(hist_v, [idx], ones)
        return _
    lax.fori_loop(0, _EW // L, _vec, None)
    tail = idx_v[pl.ds((_EW // L) * L, L)]
    lane = lax.iota(jnp.int32, L)
    tail = jnp.where(lane < (_EW % L), tail, N)
    plsc.addupdate_scatter(hist_v, [tail], ones)

    pltpu.sync_copy(hist_v, shared.at[sid])
    plsc.subcore_barrier()

    seg = sid * SEG
    pltpu.sync_copy(shared.at[:, pl.ds(seg, SEG)], blk_v)

    def _reduce(i, _):
        acc = blk_v[0, pl.ds(i * L, L)]
        for r in range(1, NS):
            acc = acc + blk_v[r, pl.ds(i * L, L)]
        red_v[pl.ds(i * L, L)] = acc
        return _
    lax.fori_loop(0, SEG // L, _reduce, None)

    pltpu.sync_copy(red_v, out_hbm.at[cid, pl.ds(seg, SEG)])


@functools.partial(
    pl.kernel,
    out_type=jax.ShapeDtypeStruct((NC, NPAD, DH), jnp.float32),
    mesh=_MESH,
    scratch_types=[
        pltpu.VMEM((NCH, CH), jnp.int32),
        pltpu.VMEM((NCH, CH), jnp.int32),
        pltpu.VMEM((CH, DH), jnp.float32),
        pltpu.VMEM((CH, DH), jnp.float32),
        pltpu.VMEM((SEG, DH), jnp.float32),
        pltpu.VMEM_SHARED((NPAD, DH), jnp.float32),
        pltpu.SemaphoreType.DMA,
        pltpu.SemaphoreType.DMA,
        pltpu.SemaphoreType.DMA,
    ],
    compiler_params=_SC_PARAMS,
)
def _spmm_kernel(u_hbm, src_hbm, dst_hbm, out_hbm, src_all, dst_all,
                 rows0, rows1, wb, acc, sem0, sem1, semi):
    cid = lax.axis_index("c")
    sid = lax.axis_index("s")
    wid = cid * NS + sid

    pltpu.async_copy(src_hbm.at[wid], src_all, semi)
    pltpu.async_copy(dst_hbm.at[wid], dst_all, semi)

    zeros = jnp.zeros((L,), jnp.float32)

    def _zrow(i, _):
        wb[i, pl.ds(0, L)] = zeros
        wb[i, pl.ds(L, L)] = zeros
        return _
    lax.fori_loop(0, SEG, _zrow, None)
    pltpu.sync_copy(wb, acc.at[pl.ds(sid * SEG, SEG), :])

    pltpu.make_async_copy(src_hbm.at[wid], src_all, semi).wait()
    pltpu.make_async_copy(dst_hbm.at[wid], dst_all, semi).wait()
    pltpu.async_copy(u_hbm.at[src_all.at[0]], rows0, sem0)
    plsc.subcore_barrier()

    def _pipe(j, _):
        c0 = 2 * j
        c1 = c0 + 1
        c2 = jnp.minimum(c0 + 2, NCH - 2)
        pltpu.make_async_copy(u_hbm.at[src_all.at[c0]], rows0, sem0).wait()
        pltpu.async_copy(u_hbm.at[src_all.at[c1]], rows1, sem1)
        pltpu.sync_copy(rows0, acc.at[dst_all.at[c0]], add=True)
        pltpu.make_async_copy(u_hbm.at[src_all.at[c1]], rows1, sem1).wait()
        pltpu.async_copy(u_hbm.at[src_all.at[c2]], rows0, sem0)
        pltpu.sync_copy(rows1, acc.at[dst_all.at[c1]], add=True)
        return _
    lax.fori_loop(0, NCH // 2, _pipe, None)
    pltpu.make_async_copy(u_hbm.at[src_all.at[0]], rows0, sem0).wait()

    plsc.subcore_barrier()
    pltpu.sync_copy(acc.at[pl.ds(sid * SEG, SEG), :], wb)
    pltpu.sync_copy(wb, out_hbm.at[cid, pl.ds(sid * SEG, SEG), :])


def _proj_body(x_ref, w_ref, d_ref, u_ref, dinv_ref):
    dinv = lax.rsqrt(d_ref[0] + d_ref[1] + 1.0)
    xw = jnp.dot(x_ref[...], w_ref[...], preferred_element_type=jnp.float32)
    u_ref[...] = xw * dinv
    dinv_ref[...] = dinv


_proj1 = pl.pallas_call(
    _proj_body,
    grid=(1,),
    in_specs=[
        pl.BlockSpec((N, N_IN), lambda i: (0, 0)),
        pl.BlockSpec((N_IN, DH), lambda i: (0, 0)),
        pl.BlockSpec((NC, N, 1), lambda i: (0, 0, 0)),
    ],
    out_specs=(pl.BlockSpec((N, DH), lambda i: (0, 0)),
               pl.BlockSpec((N, 1), lambda i: (0, 0))),
    out_shape=(jax.ShapeDtypeStruct((N, DH), jnp.float32),
               jax.ShapeDtypeStruct((N, 1), jnp.float32)),
)


def _enc_body(s_ref, u1_ref, dinv_ref, b1_ref, wcat_ref, u2_ref):
    dinv = dinv_ref[...]
    h = s_ref[0] + s_ref[1] + u1_ref[...]
    h = jnp.maximum(h * dinv + b1_ref[...], 0.0)
    hm = jnp.dot(h, wcat_ref[...], preferred_element_type=jnp.float32)
    u2_ref[...] = hm * dinv


_enc = pl.pallas_call(
    _enc_body,
    grid=(1,),
    in_specs=[
        pl.BlockSpec((NC, N, DH), lambda i: (0, 0, 0)),
        pl.BlockSpec((N, DH), lambda i: (0, 0)),
        pl.BlockSpec((N, 1), lambda i: (0, 0)),
        pl.BlockSpec((1, DH), lambda i: (0, 0)),
        pl.BlockSpec((DH, DH), lambda i: (0, 0)),
    ],
    out_specs=pl.BlockSpec((N, DH), lambda i: (0, 0)),
    out_shape=jax.ShapeDtypeStruct((N, DH), jnp.float32),
)


def _head_body(s_ref, u2_ref, dinv_ref, bcat_ref, eps_ref,
               mu_ref, ls_ref, z_ref):
    g = (s_ref[0] + s_ref[1] + u2_ref[...]) * dinv_ref[...]
    g = g + bcat_ref[...]
    mu = g[:, :DL]
    ls = g[:, DL:]
    mu_ref[...] = mu
    ls_ref[...] = ls
    z_ref[...] = mu + eps_ref[...] * jnp.exp(ls)


_head = pl.pallas_call(
    _head_body,
    grid=(1,),
    in_specs=[
        pl.BlockSpec((NC, N, DH), lambda i: (0, 0, 0)),
        pl.BlockSpec((N, DH), lambda i: (0, 0)),
        pl.BlockSpec((N, 1), lambda i: (0, 0)),
        pl.BlockSpec((1, DH), lambda i: (0, 0)),
        pl.BlockSpec((N, DL), lambda i: (0, 0)),
    ],
    out_specs=(pl.BlockSpec((N, DL), lambda i: (0, 0)),
               pl.BlockSpec((N, DL), lambda i: (0, 0)),
               pl.BlockSpec((N, DL), lambda i: (0, 0))),
    out_shape=(jax.ShapeDtypeStruct((N, DL), jnp.float32),
               jax.ShapeDtypeStruct((N, DL), jnp.float32),
               jax.ShapeDtypeStruct((N, DL), jnp.float32)),
)


_BR, _BC = 400, 10000


def _dec_body(zr_ref, zc_ref, o_ref):
    o_ref[...] = lax.dot_general(
        zr_ref[...], zc_ref[...], (((1,), (1,)), ((), ())),
        preferred_element_type=jnp.float32)


_decoder = pl.pallas_call(
    _dec_body,
    grid=(N // _BR,),
    in_specs=[
        pl.BlockSpec((_BR, DL), lambda i: (i, 0)),
        pl.BlockSpec((_BC, DL), lambda i: (0, 0)),
    ],
    out_specs=pl.BlockSpec((_BR, _BC), lambda i: (i, 0)),
    out_shape=jax.ShapeDtypeStruct((N, N), jnp.float32),
)


def kernel(x, W1, b1, Wmu, bmu, Wls, bls, edge_index):
    src = edge_index[0]
    dst = edge_index[1]
    npad = EPAD - E
    src_p = jnp.concatenate(
        [src, jnp.zeros((npad,), jnp.int32)]).reshape(NW, NCH, CH)
    dst_p = jnp.concatenate(
        [dst, jnp.full((npad,), N, jnp.int32)]).reshape(NW, NCH, CH)

    degs = _deg_kernel(dst).reshape(NC, NPAD, 1)

    u1, dinv = _proj1(x, W1, degs)

    s1 = _spmm_kernel(u1, src_p, dst_p)
    wcat = jnp.concatenate([Wmu, Wls], axis=1)
    u2 = _enc(s1, u1, dinv, b1.reshape(1, DH), wcat)

    s2 = _spmm_kernel(u2, src_p, dst_p)
    bcat = jnp.concatenate([bmu, bls]).reshape(1, DH)
    eps = jax.random.normal(jax.random.key(42), (N, DL), dtype=jnp.float32)
    mu, logstd, z = _head(s2, u2, dinv, bcat, eps)
    adj = _decoder(z, z)
    return adj, mu, logstd
